# baseline (speedup 1.0000x reference)
import jax
import jax.numpy as jnp
from jax.experimental import pallas as pl
from jax.experimental.pallas import tpu as pltpu

_N = 256
_D = 4096
_K = 5
_INV_TEMPERATURE = 1.0 / 0.07
_CHUNK = 512
_STEPS = _D // _CHUNK


def _gram(x):
    return jax.lax.dot_general(
        x, x, (((1,), (1,)), ((), ())), preferred_element_type=jnp.float32
    )


def _nncon_loss_kernel(features_ref, feat_t_g_ref, out_ref, sim_acc, adc_acc):
    i = pl.program_id(0)

    g = feat_t_g_ref[...]
    f = features_ref[...]

    @pl.when(i == 0)
    def _init():
        sim_acc[...] = _gram(g)
        adc_acc[...] = _gram(f)

    @pl.when(i > 0)
    def _accum():
        sim_acc[...] += _gram(g)
        adc_acc[...] += _gram(f)

    @pl.when(i == _STEPS - 1)
    def _finish():
        sim = sim_acc[...]
        col = jax.lax.broadcasted_iota(jnp.int32, (_N, _N), 1)

        work = sim
        mask = jnp.zeros((_N, _N), dtype=jnp.float32)
        for _ in range(_K):
            row_max = jnp.max(work, axis=1, keepdims=True)
            at_max = work == row_max
            first = jnp.min(jnp.where(at_max, col, _N), axis=1, keepdims=True)
            sel = col == first
            mask = mask + sel.astype(jnp.float32)
            work = jnp.where(sel, -jnp.inf, work)

        row = jax.lax.broadcasted_iota(jnp.int32, (_N, _N), 0)
        off_diag = (row != col).astype(jnp.float32)
        mask = mask * off_diag

        adc = adc_acc[...] * _INV_TEMPERATURE
        logits_max = jnp.max(adc, axis=1, keepdims=True)
        logits = adc - logits_max

        exp_sum = jnp.sum(jnp.exp(logits) * off_diag, axis=1, keepdims=True)
        log_prob = logits - jnp.log(exp_sum)

        msum = jnp.sum(mask, axis=1)
        denom = jnp.where(msum == 0.0, 1.0, msum)
        mean_log_prob_pos = jnp.sum(mask * log_prob, axis=1) / denom

        out_ref[...] = (-jnp.sum(mean_log_prob_pos) / _N).reshape(1, 1)


@jax.jit
def kernel(features, feat_t_g):
    out = pl.pallas_call(
        _nncon_loss_kernel,
        grid=(_STEPS,),
        in_specs=[
            pl.BlockSpec((_N, _CHUNK), lambda i: (0, i)),
            pl.BlockSpec((_N, _CHUNK), lambda i: (0, i)),
        ],
        out_specs=pl.BlockSpec((1, 1), lambda i: (0, 0)),
        out_shape=jax.ShapeDtypeStruct((1, 1), jnp.float32),
        scratch_shapes=[
            pltpu.VMEM((_N, _N), jnp.float32),
            pltpu.VMEM((_N, _N), jnp.float32),
        ],
        compiler_params=pltpu.CompilerParams(
            dimension_semantics=("arbitrary",),
        ),
    )(features, feat_t_g)
    return out[0, 0]


# speedup vs baseline: 1.4401x; 1.4401x over previous
import jax
import jax.numpy as jnp
from jax.experimental import pallas as pl
from jax.experimental.pallas import tpu as pltpu

_N = 256
_D = 4096
_K = 5
_INV_TEMPERATURE = 1.0 / 0.07
_CHUNK = 2048
_STEPS = _D // _CHUNK


def _gram(x):
    return jax.lax.dot_general(
        x, x, (((1,), (1,)), ((), ())), preferred_element_type=jnp.float32
    )


def _nncon_loss_kernel(features_ref, feat_t_g_ref, out_ref, sim_acc, adc_acc):
    i = pl.program_id(0)

    g = feat_t_g_ref[...]
    f = features_ref[...]

    @pl.when(i == 0)
    def _init():
        sim_acc[...] = _gram(g)
        adc_acc[...] = _gram(f)

    @pl.when(i > 0)
    def _accum():
        sim_acc[...] += _gram(g)
        adc_acc[...] += _gram(f)

    @pl.when(i == _STEPS - 1)
    def _finish():
        sim = sim_acc[...]
        col = jax.lax.broadcasted_iota(jnp.int32, (_N, _N), 1)

        work = sim
        mask = jnp.zeros((_N, _N), dtype=jnp.float32)
        for _ in range(_K):
            row_max = jnp.max(work, axis=1, keepdims=True)
            at_max = work == row_max
            first = jnp.min(jnp.where(at_max, col, _N), axis=1, keepdims=True)
            sel = col == first
            mask = mask + sel.astype(jnp.float32)
            work = jnp.where(sel, -jnp.inf, work)

        row = jax.lax.broadcasted_iota(jnp.int32, (_N, _N), 0)
        off_diag = (row != col).astype(jnp.float32)
        mask = mask * off_diag

        adc = adc_acc[...] * _INV_TEMPERATURE
        logits_max = jnp.max(adc, axis=1, keepdims=True)
        logits = adc - logits_max

        exp_sum = jnp.sum(jnp.exp(logits) * off_diag, axis=1, keepdims=True)
        log_prob = logits - jnp.log(exp_sum)

        msum = jnp.sum(mask, axis=1)
        denom = jnp.where(msum == 0.0, 1.0, msum)
        mean_log_prob_pos = jnp.sum(mask * log_prob, axis=1) / denom

        out_ref[...] = (-jnp.sum(mean_log_prob_pos) / _N).reshape(1, 1)


@jax.jit
def kernel(features, feat_t_g):
    out = pl.pallas_call(
        _nncon_loss_kernel,
        grid=(_STEPS,),
        in_specs=[
            pl.BlockSpec((_N, _CHUNK), lambda i: (0, i)),
            pl.BlockSpec((_N, _CHUNK), lambda i: (0, i)),
        ],
        out_specs=pl.BlockSpec((1, 1), lambda i: (0, 0)),
        out_shape=jax.ShapeDtypeStruct((1, 1), jnp.float32),
        scratch_shapes=[
            pltpu.VMEM((_N, _N), jnp.float32),
            pltpu.VMEM((_N, _N), jnp.float32),
        ],
        compiler_params=pltpu.CompilerParams(
            dimension_semantics=("arbitrary",),
        ),
    )(features, feat_t_g)
    return out[0, 0]
